# baseline (speedup 1.0000x reference)
import jax
import jax.numpy as jnp
from jax.experimental import pallas as pl
from jax.experimental.pallas import tpu as pltpu

N = 10000
F = 128
H = 128
H2 = 64
BM = 400


def _s1_body(x_ref, w1_ref, o_ref):
    o_ref[...] = jnp.dot(x_ref[...], w1_ref[...],
                         preferred_element_type=jnp.float32)


def _layer1_body(adj_ref, s1_ref, b1_ref, w2_ref, o_ref):
    acc = jnp.dot(adj_ref[...], s1_ref[...],
                  preferred_element_type=jnp.float32)
    h = jnp.maximum(acc + b1_ref[...], 0.0)
    o_ref[...] = jnp.dot(h, w2_ref[...], preferred_element_type=jnp.float32)


def _layer2_body(adj_ref, h2_ref, b2_ref, o_ref):
    acc = jnp.dot(adj_ref[...], h2_ref[...],
                  preferred_element_type=jnp.float32)
    o_ref[...] = jnp.maximum(acc + b2_ref[...], 0.0)


def kernel(x, adj, W1, b1, W2, b2):
    b1r = b1.reshape(1, H)
    b2r = b2.reshape(1, H2)

    s1 = pl.pallas_call(
        _s1_body,
        out_shape=jax.ShapeDtypeStruct((N, H), jnp.float32),
    )(x, W1)

    grid = (N // BM,)
    h2 = pl.pallas_call(
        _layer1_body,
        grid=grid,
        in_specs=[
            pl.BlockSpec((BM, N), lambda i: (i, 0)),
            pl.BlockSpec((N, H), lambda i: (0, 0)),
            pl.BlockSpec((1, H), lambda i: (0, 0)),
            pl.BlockSpec((H, H2), lambda i: (0, 0)),
        ],
        out_specs=pl.BlockSpec((BM, H2), lambda i: (i, 0)),
        out_shape=jax.ShapeDtypeStruct((N, H2), jnp.float32),
        compiler_params=pltpu.CompilerParams(
            dimension_semantics=("arbitrary",),
        ),
    )(adj, s1, b1r, W2)

    out = pl.pallas_call(
        _layer2_body,
        grid=grid,
        in_specs=[
            pl.BlockSpec((BM, N), lambda i: (i, 0)),
            pl.BlockSpec((N, H2), lambda i: (0, 0)),
            pl.BlockSpec((1, H2), lambda i: (0, 0)),
        ],
        out_specs=pl.BlockSpec((BM, H2), lambda i: (i, 0)),
        out_shape=jax.ShapeDtypeStruct((N, H2), jnp.float32),
        compiler_params=pltpu.CompilerParams(
            dimension_semantics=("arbitrary",),
        ),
    )(adj, h2, b2r)
    return out


# speedup vs baseline: 1.0489x; 1.0489x over previous
import jax
import jax.numpy as jnp
from jax.experimental import pallas as pl
from jax.experimental.pallas import tpu as pltpu

N = 10000
F = 128
H = 128
H2 = 64
BM = 400


def _body(x_ref, adj_ref, w1_ref, b1_ref, w2_ref, b2_ref, o_ref,
          s1_ref, h2_ref):
    t = pl.program_id(0)
    i = pl.program_id(1)

    @pl.when((t == 0) & (i == 0))
    def _prologue():
        s1_ref[...] = jnp.dot(x_ref[...], w1_ref[...],
                              preferred_element_type=jnp.float32)

    @pl.when(t == 0)
    def _layer1():
        acc = jnp.dot(adj_ref[...], s1_ref[...],
                      preferred_element_type=jnp.float32)
        h = jnp.maximum(acc + b1_ref[...], 0.0)
        h2_ref[pl.ds(i * BM, BM), :] = jnp.dot(
            h, w2_ref[...], preferred_element_type=jnp.float32)

    @pl.when(t == 1)
    def _layer2():
        acc = jnp.dot(adj_ref[...], h2_ref[...],
                      preferred_element_type=jnp.float32)
        o_ref[pl.ds(i * BM, BM), :] = jnp.maximum(acc + b2_ref[...], 0.0)


def kernel(x, adj, W1, b1, W2, b2):
    b1r = b1.reshape(1, H)
    b2r = b2.reshape(1, H2)
    return pl.pallas_call(
        _body,
        grid=(2, N // BM),
        in_specs=[
            pl.BlockSpec((N, F), lambda t, i: (0, 0)),
            pl.BlockSpec((BM, N), lambda t, i: (i, 0)),
            pl.BlockSpec((F, H), lambda t, i: (0, 0)),
            pl.BlockSpec((1, H), lambda t, i: (0, 0)),
            pl.BlockSpec((H, H2), lambda t, i: (0, 0)),
            pl.BlockSpec((1, H2), lambda t, i: (0, 0)),
        ],
        out_specs=pl.BlockSpec((N, H2), lambda t, i: (0, 0)),
        out_shape=jax.ShapeDtypeStruct((N, H2), jnp.float32),
        scratch_shapes=[
            pltpu.VMEM((N, H), jnp.float32),
            pltpu.VMEM((N, H2), jnp.float32),
        ],
        compiler_params=pltpu.CompilerParams(
            dimension_semantics=("arbitrary", "arbitrary"),
        ),
    )(x, adj, W1, b1r, W2, b2r)
